# baseline (speedup 1.0000x reference)
import dataclasses
import functools

import jax
import jax.numpy as jnp
from jax import lax
from jax.experimental import pallas as pl
from jax.experimental.pallas import tpu as pltpu
from jax.experimental.pallas import tpu_sc as plsc

N = 10000
E = 320000
D = 128
H = 128
G = 16

NC = 2
NS = 16
NW = NC * NS
ET = E // NW
K = 128
ETP = 10240
CH = ETP // K
NBUF = 4
NP = 10240
RPT = NP // NS
BM = 1000

_mesh = plsc.VectorSubcoreMesh(
    core_axis_name="c", subcore_axis_name="s", num_cores=NC, num_subcores=NS)

_sc_params = pltpu.CompilerParams(use_tc_tiling_on_sc=False)
if "needs_layout_passes" in pltpu.CompilerParams.__dataclass_fields__:
    _sc_params = dataclasses.replace(_sc_params, needs_layout_passes=False)


def _splat16(v):
    return jnp.full((16,), v, jnp.int32)


@functools.partial(
    pl.kernel,
    out_type=jax.ShapeDtypeStruct((NC, NP, 16), jnp.float32),
    mesh=_mesh,
    scratch_types=[
        pltpu.VMEM((CH, K), jnp.int32),
        pltpu.VMEM((CH, K), jnp.float32),
        pltpu.VMEM((K, 16), jnp.float32),
        pltpu.VMEM_SHARED((NP, 16), jnp.float32),
    ],
    compiler_params=_sc_params,
)
def _deg_kernel(dst_hbm, w_hbm, zer_hbm, out_hbm, dstv, wv, wrow, acc):
    cid = lax.axis_index("c")
    sid = lax.axis_index("s")
    wid = cid * NS + sid
    pltpu.sync_copy(zer_hbm, acc.at[pl.ds(sid * RPT, RPT)])
    pltpu.sync_copy(dst_hbm.at[wid], dstv)
    pltpu.sync_copy(w_hbm.at[wid], wv)
    plsc.subcore_barrier()

    @pl.loop(0, CH)
    def _chunk(j):
        @pl.loop(0, K)
        def _row(i):
            wb = plsc.load_gather(wv, [_splat16(j), _splat16(i)])
            wrow[i, pl.ds(0, 16)] = wb
        pltpu.sync_copy(wrow, acc.at[dstv.at[j]], add=True)

    plsc.subcore_barrier()
    pltpu.sync_copy(acc.at[pl.ds(sid * RPT, RPT)],
                    out_hbm.at[cid, pl.ds(sid * RPT, RPT)])


DH = D // 2

@functools.partial(
    pl.kernel,
    out_type=jax.ShapeDtypeStruct((NC, 2, NP, DH), jnp.float32),
    mesh=_mesh,
    scratch_types=(
        [pltpu.VMEM((CH, K), jnp.int32),
         pltpu.VMEM((CH, K), jnp.int32),
         pltpu.VMEM((CH, K), jnp.float32)]
        + [pltpu.VMEM((K, DH), jnp.float32) for _ in range(NBUF)]
        + [pltpu.VMEM_SHARED((NP, DH), jnp.float32)]
        + [pltpu.SemaphoreType.DMA for _ in range(2 * NBUF)]
    ),
    compiler_params=_sc_params,
)
def _agg_kernel(hs_lo_hbm, hs_hi_hbm, src_hbm, dst_hbm, w_hbm, zer_hbm,
                out_hbm, srcv, dstv, wv, r0, r1, r2, r3, acc,
                g0, g1, g2, g3, s0, s1, s2, s3):
    rows = [r0, r1, r2, r3]
    gsem = [g0, g1, g2, g3]
    ssem = [s0, s1, s2, s3]
    cid = lax.axis_index("c")
    sid = lax.axis_index("s")
    wid = cid * NS + sid
    pltpu.sync_copy(src_hbm.at[wid], srcv)
    pltpu.sync_copy(dst_hbm.at[wid], dstv)
    pltpu.sync_copy(w_hbm.at[wid], wv)

    for phase, hs_hbm in enumerate([hs_lo_hbm, hs_hi_hbm]):
        pltpu.sync_copy(zer_hbm, acc.at[pl.ds(sid * RPT, RPT)])
        plsc.subcore_barrier()

        @pl.loop(0, CH // NBUF)
        def _grp(t):
            j0 = t * NBUF
            gd = [pltpu.async_copy(hs_hbm.at[srcv.at[j0 + u]], rows[u],
                                   gsem[u])
                  for u in range(NBUF)]
            sd = []
            for u in range(NBUF):
                gd[u].wait()
                jb = _splat16(j0 + u)
                @pl.loop(0, K, step=4)
                def _row(i, _u=u, _jb=jb):
                    for q in range(4):
                        wb = plsc.load_gather(wv, [_jb, _splat16(i + q)])
                        for s in range(DH // 16):
                            sl = (i + q, pl.ds(s * 16, 16))
                            rows[_u][sl] = rows[_u][sl] * wb
                sd.append(pltpu.async_copy(rows[u], acc.at[dstv.at[j0 + u]],
                                           ssem[u], add=True))
            for d in sd:
                d.wait()

        plsc.subcore_barrier()
        pltpu.sync_copy(acc.at[pl.ds(sid * RPT, RPT)],
                        out_hbm.at[cid, phase, pl.ds(sid * RPT, RPT)])


def _mm_body(x_ref, w_ref, o_ref):
    o_ref[...] = jnp.dot(x_ref[...], w_ref[...],
                         preferred_element_type=jnp.float32)


def _dis_scale_body(degp_ref, h_ref, dis_ref, hs_ref):
    dp = degp_ref[...]
    deg = dp[0, :, 0:1] + dp[1, :, 0:1] + 1.0
    dis = lax.rsqrt(deg)
    dis_ref[...] = dis
    hs_ref[...] = h_ref[...] * dis


def _layer2_body(p_ref, hs1_ref, dis_ref, b1_ref, w2_ref, hs2_ref):
    p = p_ref[...]
    ps = p[0] + p[1]
    pcat = jnp.concatenate([ps[0], ps[1]], axis=1)
    dis = dis_ref[...]
    x2 = jnp.maximum((pcat + hs1_ref[...]) * dis + b1_ref[...], 0.0)
    hs2_ref[...] = jnp.dot(x2, w2_ref[...],
                           preferred_element_type=jnp.float32) * dis


def _head_body(p_ref, hs2_ref, dis_ref, b2_ref, batch_ref,
               fc1w_ref, fc1b_ref, fc2w_ref, fc2b_ref, o_ref,
               sum_ref, max_ref, cnt_ref):
    i = pl.program_id(0)

    @pl.when(i == 0)
    def _init():
        sum_ref[...] = jnp.zeros_like(sum_ref)
        max_ref[...] = jnp.full_like(max_ref, -jnp.inf)
        cnt_ref[...] = jnp.zeros_like(cnt_ref)

    p = p_ref[...]
    ps = p[0] + p[1]
    pcat = jnp.concatenate([ps[0], ps[1]], axis=1)
    dis = dis_ref[...]
    x3 = jnp.maximum((pcat + hs2_ref[...]) * dis + b2_ref[...], 0.0)
    bt = batch_ref[...]
    for g in range(G):
        m = bt == g
        cnt_ref[g:g + 1, :] += jnp.sum(m.astype(jnp.float32))
        sum_ref[g:g + 1, :] += jnp.sum(jnp.where(m, x3, 0.0), axis=0,
                                       keepdims=True)
        max_ref[g:g + 1, :] = jnp.maximum(
            max_ref[g:g + 1, :],
            jnp.max(jnp.where(m, x3, -jnp.inf), axis=0, keepdims=True))

    @pl.when(i == N // BM - 1)
    def _fin():
        mean = sum_ref[...] / jnp.maximum(cnt_ref[...], 1.0)
        gf = jnp.concatenate([mean, max_ref[...]], axis=1)
        r = jnp.maximum(jnp.dot(gf, fc1w_ref[...],
                                preferred_element_type=jnp.float32)
                        + fc1b_ref[...], 0.0)
        o_ref[...] = jnp.dot(r, fc2w_ref[...],
                             preferred_element_type=jnp.float32) + fc2b_ref[...]


def kernel(x, edge_index, edge_attr, batch, W1, b1, W2, b2,
           fc1_w, fc1_b, fc2_w, fc2_b):
    f32 = jnp.float32
    pad = ((0, 0), (0, ETP - ET))
    src3 = jnp.pad(edge_index[0].astype(jnp.int32).reshape(NW, ET),
                   pad).reshape(NW, CH, K)
    dst3 = jnp.pad(edge_index[1].astype(jnp.int32).reshape(NW, ET),
                   pad).reshape(NW, CH, K)
    w3 = jnp.pad(edge_attr.astype(f32).reshape(NW, ET), pad).reshape(NW, CH, K)
    batch2 = batch.astype(jnp.int32).reshape(N, 1)
    zrow = jnp.zeros((RPT, DH), f32)
    zdeg = jnp.zeros((RPT, 16), f32)

    b1r = b1.reshape(1, H)
    b2r = b2.reshape(1, H)
    fc1_br = fc1_b.reshape(1, 1024)
    fc2_br = fc2_b.reshape(1, 1)

    degp = _deg_kernel(dst3, w3, zdeg)

    h1 = pl.pallas_call(
        _mm_body,
        grid=(N // BM,),
        in_specs=[pl.BlockSpec((BM, D), lambda i: (i, 0)),
                  pl.BlockSpec((D, H), lambda i: (0, 0))],
        out_specs=pl.BlockSpec((BM, H), lambda i: (i, 0)),
        out_shape=jax.ShapeDtypeStruct((N, H), f32),
    )(x, W1)

    dis, hs1 = pl.pallas_call(
        _dis_scale_body,
        grid=(N // BM,),
        in_specs=[pl.BlockSpec((NC, BM, 16), lambda i: (0, i, 0)),
                  pl.BlockSpec((BM, H), lambda i: (i, 0))],
        out_specs=[pl.BlockSpec((BM, 1), lambda i: (i, 0)),
                   pl.BlockSpec((BM, H), lambda i: (i, 0))],
        out_shape=[jax.ShapeDtypeStruct((N, 1), f32),
                   jax.ShapeDtypeStruct((N, H), f32)],
    )(degp, h1)

    p1 = _agg_kernel(hs1[:, :DH], hs1[:, DH:], src3, dst3, w3, zrow)

    hs2 = pl.pallas_call(
        _layer2_body,
        grid=(N // BM,),
        in_specs=[pl.BlockSpec((NC, 2, BM, DH), lambda i: (0, 0, i, 0)),
                  pl.BlockSpec((BM, H), lambda i: (i, 0)),
                  pl.BlockSpec((BM, 1), lambda i: (i, 0)),
                  pl.BlockSpec((1, H), lambda i: (0, 0)),
                  pl.BlockSpec((H, H), lambda i: (0, 0))],
        out_specs=pl.BlockSpec((BM, H), lambda i: (i, 0)),
        out_shape=jax.ShapeDtypeStruct((N, H), f32),
    )(p1, hs1, dis, b1r, W2)

    p2 = _agg_kernel(hs2[:, :DH], hs2[:, DH:], src3, dst3, w3, zrow)

    out = pl.pallas_call(
        _head_body,
        grid=(N // BM,),
        in_specs=[pl.BlockSpec((NC, 2, BM, DH), lambda i: (0, 0, i, 0)),
                  pl.BlockSpec((BM, H), lambda i: (i, 0)),
                  pl.BlockSpec((BM, 1), lambda i: (i, 0)),
                  pl.BlockSpec((1, H), lambda i: (0, 0)),
                  pl.BlockSpec((BM, 1), lambda i: (i, 0)),
                  pl.BlockSpec((2 * H, 1024), lambda i: (0, 0)),
                  pl.BlockSpec((1, 1024), lambda i: (0, 0)),
                  pl.BlockSpec((1024, 1), lambda i: (0, 0)),
                  pl.BlockSpec((1, 1), lambda i: (0, 0))],
        out_specs=pl.BlockSpec((G, 1), lambda i: (0, 0)),
        out_shape=jax.ShapeDtypeStruct((G, 1), f32),
        scratch_shapes=[pltpu.VMEM((G, H), f32),
                        pltpu.VMEM((G, H), f32),
                        pltpu.VMEM((G, H), f32)],
    )(p2, hs2, dis, b2r, batch2, fc1_w, fc1_br, fc2_w, fc2_br)
    return out


# speedup vs baseline: 1.0402x; 1.0402x over previous
import dataclasses
import functools

import jax
import jax.numpy as jnp
from jax import lax
from jax.experimental import pallas as pl
from jax.experimental.pallas import tpu as pltpu
from jax.experimental.pallas import tpu_sc as plsc

N = 10000
E = 320000
D = 128
H = 128
G = 16

NC = 2
NS = 16
NW = NC * NS
ET = E // NW
K = 128
ETP = 10240
CH = ETP // K
NBUF = 4
NP = 10240
RPT = NP // NS
BM = 1000

_mesh = plsc.VectorSubcoreMesh(
    core_axis_name="c", subcore_axis_name="s", num_cores=NC, num_subcores=NS)

_sc_params = pltpu.CompilerParams(use_tc_tiling_on_sc=False)
if "needs_layout_passes" in pltpu.CompilerParams.__dataclass_fields__:
    _sc_params = dataclasses.replace(_sc_params, needs_layout_passes=False)


def _splat16(v):
    return jnp.full((16,), v, jnp.int32)


@functools.partial(
    pl.kernel,
    out_type=jax.ShapeDtypeStruct((NC, NP, 16), jnp.float32),
    mesh=_mesh,
    scratch_types=[
        pltpu.VMEM((CH, K), jnp.int32),
        pltpu.VMEM((CH, K), jnp.float32),
        pltpu.VMEM((K, 16), jnp.float32),
        pltpu.VMEM_SHARED((NP, 16), jnp.float32),
    ],
    compiler_params=_sc_params,
)
def _deg_kernel(dst_hbm, w_hbm, zer_hbm, out_hbm, dstv, wv, wrow, acc):
    cid = lax.axis_index("c")
    sid = lax.axis_index("s")
    wid = cid * NS + sid
    pltpu.sync_copy(zer_hbm, acc.at[pl.ds(sid * RPT, RPT)])
    pltpu.sync_copy(dst_hbm.at[wid], dstv)
    pltpu.sync_copy(w_hbm.at[wid], wv)
    plsc.subcore_barrier()

    @pl.loop(0, CH)
    def _chunk(j):
        @pl.loop(0, K)
        def _row(i):
            wb = plsc.load_gather(wv, [_splat16(j), _splat16(i)])
            wrow[i, pl.ds(0, 16)] = wb
        pltpu.sync_copy(wrow, acc.at[dstv.at[j]], add=True)

    plsc.subcore_barrier()
    pltpu.sync_copy(acc.at[pl.ds(sid * RPT, RPT)],
                    out_hbm.at[cid, pl.ds(sid * RPT, RPT)])


DH = D // 2

@functools.partial(
    pl.kernel,
    out_type=jax.ShapeDtypeStruct((NC, 2, NP, DH), jnp.float32),
    mesh=_mesh,
    scratch_types=(
        [pltpu.VMEM((CH, K), jnp.int32),
         pltpu.VMEM((CH, K), jnp.int32),
         pltpu.VMEM((CH, K), jnp.float32)]
        + [pltpu.VMEM((K, DH), jnp.float32) for _ in range(2)]
        + [pltpu.VMEM_SHARED((NP, DH), jnp.float32)]
        + [pltpu.SemaphoreType.DMA for _ in range(2)]
    ),
    compiler_params=_sc_params,
)
def _agg_kernel(hs_lo_hbm, hs_hi_hbm, src_hbm, dst_hbm, w_hbm, zer_hbm,
                out_hbm, srcv, dstv, wv, r0, r1, acc, g0, g1):
    rows = [r0, r1]
    gsem = [g0, g1]
    cid = lax.axis_index("c")
    sid = lax.axis_index("s")
    wid = cid * NS + sid
    pltpu.sync_copy(src_hbm.at[wid], srcv)
    pltpu.sync_copy(dst_hbm.at[wid], dstv)
    pltpu.sync_copy(w_hbm.at[wid], wv)

    def _scale(buf, j):
        jb = _splat16(j)

        @pl.loop(0, K, step=4)
        def _rg(i):
            for q in range(4):
                wb = plsc.load_gather(wv, [jb, _splat16(i + q)])
                for s in range(DH // 16):
                    sl = (i + q, pl.ds(s * 16, 16))
                    buf[sl] = buf[sl] * wb

    for phase, hs_hbm in enumerate([hs_lo_hbm, hs_hi_hbm]):
        pltpu.sync_copy(zer_hbm, acc.at[pl.ds(sid * RPT, RPT)])
        plsc.subcore_barrier()

        pltpu.async_copy(hs_hbm.at[srcv.at[0]], rows[0], gsem[0])

        @pl.loop(0, CH, step=2)
        def _grp(t):
            pltpu.async_copy(hs_hbm.at[srcv.at[t + 1]], rows[1], gsem[1])
            pltpu.make_async_copy(hs_hbm.at[srcv.at[t]], rows[0],
                                  gsem[0]).wait()
            _scale(rows[0], t)
            pltpu.sync_copy(rows[0], acc.at[dstv.at[t]], add=True)

            @pl.when(t + 2 < CH)
            def _nxt():
                pltpu.async_copy(hs_hbm.at[srcv.at[t + 2]], rows[0], gsem[0])

            pltpu.make_async_copy(hs_hbm.at[srcv.at[t + 1]], rows[1],
                                  gsem[1]).wait()
            _scale(rows[1], t + 1)
            pltpu.sync_copy(rows[1], acc.at[dstv.at[t + 1]], add=True)

        plsc.subcore_barrier()
        pltpu.sync_copy(acc.at[pl.ds(sid * RPT, RPT)],
                        out_hbm.at[cid, phase, pl.ds(sid * RPT, RPT)])


def _mm_body(x_ref, w_ref, o_ref):
    o_ref[...] = jnp.dot(x_ref[...], w_ref[...],
                         preferred_element_type=jnp.float32)


def _dis_scale_body(degp_ref, h_ref, dis_ref, hs_ref):
    dp = degp_ref[...]
    deg = dp[0, :, 0:1] + dp[1, :, 0:1] + 1.0
    dis = lax.rsqrt(deg)
    dis_ref[...] = dis
    hs_ref[...] = h_ref[...] * dis


def _layer2_body(p_ref, hs1_ref, dis_ref, b1_ref, w2_ref, hs2_ref):
    p = p_ref[...]
    ps = p[0] + p[1]
    pcat = jnp.concatenate([ps[0], ps[1]], axis=1)
    dis = dis_ref[...]
    x2 = jnp.maximum((pcat + hs1_ref[...]) * dis + b1_ref[...], 0.0)
    hs2_ref[...] = jnp.dot(x2, w2_ref[...],
                           preferred_element_type=jnp.float32) * dis


def _head_body(p_ref, hs2_ref, dis_ref, b2_ref, batch_ref,
               fc1w_ref, fc1b_ref, fc2w_ref, fc2b_ref, o_ref,
               sum_ref, max_ref, cnt_ref):
    i = pl.program_id(0)

    @pl.when(i == 0)
    def _init():
        sum_ref[...] = jnp.zeros_like(sum_ref)
        max_ref[...] = jnp.full_like(max_ref, -jnp.inf)
        cnt_ref[...] = jnp.zeros_like(cnt_ref)

    p = p_ref[...]
    ps = p[0] + p[1]
    pcat = jnp.concatenate([ps[0], ps[1]], axis=1)
    dis = dis_ref[...]
    x3 = jnp.maximum((pcat + hs2_ref[...]) * dis + b2_ref[...], 0.0)
    bt = batch_ref[...]
    for g in range(G):
        m = bt == g
        cnt_ref[g:g + 1, :] += jnp.sum(m.astype(jnp.float32))
        sum_ref[g:g + 1, :] += jnp.sum(jnp.where(m, x3, 0.0), axis=0,
                                       keepdims=True)
        max_ref[g:g + 1, :] = jnp.maximum(
            max_ref[g:g + 1, :],
            jnp.max(jnp.where(m, x3, -jnp.inf), axis=0, keepdims=True))

    @pl.when(i == N // BM - 1)
    def _fin():
        mean = sum_ref[...] / jnp.maximum(cnt_ref[...], 1.0)
        gf = jnp.concatenate([mean, max_ref[...]], axis=1)
        r = jnp.maximum(jnp.dot(gf, fc1w_ref[...],
                                preferred_element_type=jnp.float32)
                        + fc1b_ref[...], 0.0)
        o_ref[...] = jnp.dot(r, fc2w_ref[...],
                             preferred_element_type=jnp.float32) + fc2b_ref[...]


def kernel(x, edge_index, edge_attr, batch, W1, b1, W2, b2,
           fc1_w, fc1_b, fc2_w, fc2_b):
    f32 = jnp.float32
    pad = ((0, 0), (0, ETP - ET))
    src3 = jnp.pad(edge_index[0].astype(jnp.int32).reshape(NW, ET),
                   pad).reshape(NW, CH, K)
    dst3 = jnp.pad(edge_index[1].astype(jnp.int32).reshape(NW, ET),
                   pad).reshape(NW, CH, K)
    w3 = jnp.pad(edge_attr.astype(f32).reshape(NW, ET), pad).reshape(NW, CH, K)
    batch2 = batch.astype(jnp.int32).reshape(N, 1)
    zrow = jnp.zeros((RPT, DH), f32)
    zdeg = jnp.zeros((RPT, 16), f32)

    b1r = b1.reshape(1, H)
    b2r = b2.reshape(1, H)
    fc1_br = fc1_b.reshape(1, 1024)
    fc2_br = fc2_b.reshape(1, 1)

    degp = _deg_kernel(dst3, w3, zdeg)

    h1 = pl.pallas_call(
        _mm_body,
        grid=(N // BM,),
        in_specs=[pl.BlockSpec((BM, D), lambda i: (i, 0)),
                  pl.BlockSpec((D, H), lambda i: (0, 0))],
        out_specs=pl.BlockSpec((BM, H), lambda i: (i, 0)),
        out_shape=jax.ShapeDtypeStruct((N, H), f32),
    )(x, W1)

    dis, hs1 = pl.pallas_call(
        _dis_scale_body,
        grid=(N // BM,),
        in_specs=[pl.BlockSpec((NC, BM, 16), lambda i: (0, i, 0)),
                  pl.BlockSpec((BM, H), lambda i: (i, 0))],
        out_specs=[pl.BlockSpec((BM, 1), lambda i: (i, 0)),
                   pl.BlockSpec((BM, H), lambda i: (i, 0))],
        out_shape=[jax.ShapeDtypeStruct((N, 1), f32),
                   jax.ShapeDtypeStruct((N, H), f32)],
    )(degp, h1)

    p1 = _agg_kernel(hs1[:, :DH], hs1[:, DH:], src3, dst3, w3, zrow)

    hs2 = pl.pallas_call(
        _layer2_body,
        grid=(N // BM,),
        in_specs=[pl.BlockSpec((NC, 2, BM, DH), lambda i: (0, 0, i, 0)),
                  pl.BlockSpec((BM, H), lambda i: (i, 0)),
                  pl.BlockSpec((BM, 1), lambda i: (i, 0)),
                  pl.BlockSpec((1, H), lambda i: (0, 0)),
                  pl.BlockSpec((H, H), lambda i: (0, 0))],
        out_specs=pl.BlockSpec((BM, H), lambda i: (i, 0)),
        out_shape=jax.ShapeDtypeStruct((N, H), f32),
    )(p1, hs1, dis, b1r, W2)

    p2 = _agg_kernel(hs2[:, :DH], hs2[:, DH:], src3, dst3, w3, zrow)

    out = pl.pallas_call(
        _head_body,
        grid=(N // BM,),
        in_specs=[pl.BlockSpec((NC, 2, BM, DH), lambda i: (0, 0, i, 0)),
                  pl.BlockSpec((BM, H), lambda i: (i, 0)),
                  pl.BlockSpec((BM, 1), lambda i: (i, 0)),
                  pl.BlockSpec((1, H), lambda i: (0, 0)),
                  pl.BlockSpec((BM, 1), lambda i: (i, 0)),
                  pl.BlockSpec((2 * H, 1024), lambda i: (0, 0)),
                  pl.BlockSpec((1, 1024), lambda i: (0, 0)),
                  pl.BlockSpec((1024, 1), lambda i: (0, 0)),
                  pl.BlockSpec((1, 1), lambda i: (0, 0))],
        out_specs=pl.BlockSpec((G, 1), lambda i: (0, 0)),
        out_shape=jax.ShapeDtypeStruct((G, 1), f32),
        scratch_shapes=[pltpu.VMEM((G, H), f32),
                        pltpu.VMEM((G, H), f32),
                        pltpu.VMEM((G, H), f32)],
    )(p2, hs2, dis, b2r, batch2, fc1_w, fc1_br, fc2_w, fc2_br)
    return out


# speedup vs baseline: 1.0666x; 1.0254x over previous
import dataclasses
import functools

import jax
import jax.numpy as jnp
from jax import lax
from jax.experimental import pallas as pl
from jax.experimental.pallas import tpu as pltpu
from jax.experimental.pallas import tpu_sc as plsc

N = 10000
E = 320000
D = 128
H = 128
G = 16

NC = 2
NS = 16
NW = NC * NS
ET = E // NW
K = 128
ETP = 10240
CH = ETP // K
NBUF = 4
NP = 10240
RPT = NP // NS
BM = 1000

_mesh = plsc.VectorSubcoreMesh(
    core_axis_name="c", subcore_axis_name="s", num_cores=NC, num_subcores=NS)

_sc_params = pltpu.CompilerParams(use_tc_tiling_on_sc=False)
if "needs_layout_passes" in pltpu.CompilerParams.__dataclass_fields__:
    _sc_params = dataclasses.replace(_sc_params, needs_layout_passes=False)


def _splat16(v):
    return jnp.full((16,), v, jnp.int32)


def _bf16_table(hs_half):
    n = hs_half.shape[0]
    t = hs_half.reshape(n, DH // 32, 2, 16).transpose(0, 1, 3, 2)
    return t.reshape(n, DH).astype(jnp.bfloat16)


@functools.partial(
    pl.kernel,
    out_type=jax.ShapeDtypeStruct((NC, NP, 16), jnp.float32),
    mesh=_mesh,
    scratch_types=[
        pltpu.VMEM((CH, K), jnp.int32),
        pltpu.VMEM((CH, K), jnp.float32),
        pltpu.VMEM((K, 16), jnp.float32),
        pltpu.VMEM_SHARED((NP, 16), jnp.float32),
    ],
    compiler_params=_sc_params,
)
def _deg_kernel(dst_hbm, w_hbm, zer_hbm, out_hbm, dstv, wv, wrow, acc):
    cid = lax.axis_index("c")
    sid = lax.axis_index("s")
    wid = cid * NS + sid
    pltpu.sync_copy(zer_hbm, acc.at[pl.ds(sid * RPT, RPT)])
    pltpu.sync_copy(dst_hbm.at[wid], dstv)
    pltpu.sync_copy(w_hbm.at[wid], wv)
    plsc.subcore_barrier()

    @pl.loop(0, CH)
    def _chunk(j):
        @pl.loop(0, K)
        def _row(i):
            wb = plsc.load_gather(wv, [_splat16(j), _splat16(i)])
            wrow[i, pl.ds(0, 16)] = wb
        pltpu.sync_copy(wrow, acc.at[dstv.at[j]], add=True)

    plsc.subcore_barrier()
    pltpu.sync_copy(acc.at[pl.ds(sid * RPT, RPT)],
                    out_hbm.at[cid, pl.ds(sid * RPT, RPT)])


DH = D // 2

@functools.partial(
    pl.kernel,
    out_type=jax.ShapeDtypeStruct((NC, 2, NP, DH), jnp.float32),
    mesh=_mesh,
    scratch_types=(
        [pltpu.VMEM((CH, K), jnp.int32),
         pltpu.VMEM((CH, K), jnp.int32),
         pltpu.VMEM((CH, K), jnp.float32)]
        + [pltpu.VMEM((K, DH), jnp.bfloat16) for _ in range(2)]
        + [pltpu.VMEM((K, DH), jnp.float32)]
        + [pltpu.VMEM_SHARED((NP, DH), jnp.float32)]
        + [pltpu.SemaphoreType.DMA for _ in range(2)]
    ),
    compiler_params=_sc_params,
)
def _agg_kernel(hs_lo_hbm, hs_hi_hbm, src_hbm, dst_hbm, w_hbm, zer_hbm,
                out_hbm, srcv, dstv, wv, rb0, rb1, rf, acc, g0, g1):
    rows = [rb0, rb1]
    gsem = [g0, g1]
    cid = lax.axis_index("c")
    sid = lax.axis_index("s")
    wid = cid * NS + sid
    pltpu.sync_copy(src_hbm.at[wid], srcv)
    pltpu.sync_copy(dst_hbm.at[wid], dstv)
    pltpu.sync_copy(w_hbm.at[wid], wv)

    def _scale(buf, j):
        jb = _splat16(j)

        @pl.loop(0, K, step=2)
        def _rg(i):
            for q in range(2):
                wb = plsc.load_gather(wv, [jb, _splat16(i + q)])
                for c in range(DH // 32):
                    m = buf[i + q, pl.ds(32 * c, 32)]
                    a, b = plsc.unpack(m, format=plsc.PackFormat.INTERLEAVED)
                    rf[i + q, pl.ds(32 * c, 16)] = a * wb
                    rf[i + q, pl.ds(32 * c + 16, 16)] = b * wb

    for phase, hs_hbm in enumerate([hs_lo_hbm, hs_hi_hbm]):
        pltpu.sync_copy(zer_hbm, acc.at[pl.ds(sid * RPT, RPT)])
        plsc.subcore_barrier()

        pltpu.async_copy(hs_hbm.at[srcv.at[0]], rows[0], gsem[0])

        @pl.loop(0, CH, step=2)
        def _grp(t):
            pltpu.async_copy(hs_hbm.at[srcv.at[t + 1]], rows[1], gsem[1])
            pltpu.make_async_copy(hs_hbm.at[srcv.at[t]], rows[0],
                                  gsem[0]).wait()
            _scale(rows[0], t)
            pltpu.sync_copy(rf, acc.at[dstv.at[t]], add=True)

            @pl.when(t + 2 < CH)
            def _nxt():
                pltpu.async_copy(hs_hbm.at[srcv.at[t + 2]], rows[0], gsem[0])

            pltpu.make_async_copy(hs_hbm.at[srcv.at[t + 1]], rows[1],
                                  gsem[1]).wait()
            _scale(rows[1], t + 1)
            pltpu.sync_copy(rf, acc.at[dstv.at[t + 1]], add=True)

        plsc.subcore_barrier()
        pltpu.sync_copy(acc.at[pl.ds(sid * RPT, RPT)],
                        out_hbm.at[cid, phase, pl.ds(sid * RPT, RPT)])


def _mm_body(x_ref, w_ref, o_ref):
    o_ref[...] = jnp.dot(x_ref[...], w_ref[...],
                         preferred_element_type=jnp.float32)


def _dis_scale_body(degp_ref, h_ref, dis_ref, hs_ref):
    dp = degp_ref[...]
    deg = dp[0, :, 0:1] + dp[1, :, 0:1] + 1.0
    dis = lax.rsqrt(deg)
    dis_ref[...] = dis
    hs_ref[...] = h_ref[...] * dis


def _layer2_body(p_ref, hs1_ref, dis_ref, b1_ref, w2_ref, hs2_ref):
    p = p_ref[...]
    ps = p[0] + p[1]
    pcat = jnp.concatenate([ps[0], ps[1]], axis=1)
    dis = dis_ref[...]
    x2 = jnp.maximum((pcat + hs1_ref[...]) * dis + b1_ref[...], 0.0)
    hs2_ref[...] = jnp.dot(x2, w2_ref[...],
                           preferred_element_type=jnp.float32) * dis


def _head_body(p_ref, hs2_ref, dis_ref, b2_ref, batch_ref,
               fc1w_ref, fc1b_ref, fc2w_ref, fc2b_ref, o_ref,
               sum_ref, max_ref, cnt_ref):
    i = pl.program_id(0)

    @pl.when(i == 0)
    def _init():
        sum_ref[...] = jnp.zeros_like(sum_ref)
        max_ref[...] = jnp.full_like(max_ref, -jnp.inf)
        cnt_ref[...] = jnp.zeros_like(cnt_ref)

    p = p_ref[...]
    ps = p[0] + p[1]
    pcat = jnp.concatenate([ps[0], ps[1]], axis=1)
    dis = dis_ref[...]
    x3 = jnp.maximum((pcat + hs2_ref[...]) * dis + b2_ref[...], 0.0)
    bt = batch_ref[...]
    for g in range(G):
        m = bt == g
        cnt_ref[g:g + 1, :] += jnp.sum(m.astype(jnp.float32))
        sum_ref[g:g + 1, :] += jnp.sum(jnp.where(m, x3, 0.0), axis=0,
                                       keepdims=True)
        max_ref[g:g + 1, :] = jnp.maximum(
            max_ref[g:g + 1, :],
            jnp.max(jnp.where(m, x3, -jnp.inf), axis=0, keepdims=True))

    @pl.when(i == N // BM - 1)
    def _fin():
        mean = sum_ref[...] / jnp.maximum(cnt_ref[...], 1.0)
        gf = jnp.concatenate([mean, max_ref[...]], axis=1)
        r = jnp.maximum(jnp.dot(gf, fc1w_ref[...],
                                preferred_element_type=jnp.float32)
                        + fc1b_ref[...], 0.0)
        o_ref[...] = jnp.dot(r, fc2w_ref[...],
                             preferred_element_type=jnp.float32) + fc2b_ref[...]


def kernel(x, edge_index, edge_attr, batch, W1, b1, W2, b2,
           fc1_w, fc1_b, fc2_w, fc2_b):
    f32 = jnp.float32
    pad = ((0, 0), (0, ETP - ET))
    src3 = jnp.pad(edge_index[0].astype(jnp.int32).reshape(NW, ET),
                   pad).reshape(NW, CH, K)
    dst3 = jnp.pad(edge_index[1].astype(jnp.int32).reshape(NW, ET),
                   pad).reshape(NW, CH, K)
    w3 = jnp.pad(edge_attr.astype(f32).reshape(NW, ET), pad).reshape(NW, CH, K)
    batch2 = batch.astype(jnp.int32).reshape(N, 1)
    zrow = jnp.zeros((RPT, DH), f32)
    zdeg = jnp.zeros((RPT, 16), f32)

    b1r = b1.reshape(1, H)
    b2r = b2.reshape(1, H)
    fc1_br = fc1_b.reshape(1, 1024)
    fc2_br = fc2_b.reshape(1, 1)

    degp = _deg_kernel(dst3, w3, zdeg)

    h1 = pl.pallas_call(
        _mm_body,
        grid=(N // BM,),
        in_specs=[pl.BlockSpec((BM, D), lambda i: (i, 0)),
                  pl.BlockSpec((D, H), lambda i: (0, 0))],
        out_specs=pl.BlockSpec((BM, H), lambda i: (i, 0)),
        out_shape=jax.ShapeDtypeStruct((N, H), f32),
    )(x, W1)

    dis, hs1 = pl.pallas_call(
        _dis_scale_body,
        grid=(N // BM,),
        in_specs=[pl.BlockSpec((NC, BM, 16), lambda i: (0, i, 0)),
                  pl.BlockSpec((BM, H), lambda i: (i, 0))],
        out_specs=[pl.BlockSpec((BM, 1), lambda i: (i, 0)),
                   pl.BlockSpec((BM, H), lambda i: (i, 0))],
        out_shape=[jax.ShapeDtypeStruct((N, 1), f32),
                   jax.ShapeDtypeStruct((N, H), f32)],
    )(degp, h1)

    p1 = _agg_kernel(_bf16_table(hs1[:, :DH]), _bf16_table(hs1[:, DH:]),
                     src3, dst3, w3, zrow)

    hs2 = pl.pallas_call(
        _layer2_body,
        grid=(N // BM,),
        in_specs=[pl.BlockSpec((NC, 2, BM, DH), lambda i: (0, 0, i, 0)),
                  pl.BlockSpec((BM, H), lambda i: (i, 0)),
                  pl.BlockSpec((BM, 1), lambda i: (i, 0)),
                  pl.BlockSpec((1, H), lambda i: (0, 0)),
                  pl.BlockSpec((H, H), lambda i: (0, 0))],
        out_specs=pl.BlockSpec((BM, H), lambda i: (i, 0)),
        out_shape=jax.ShapeDtypeStruct((N, H), f32),
    )(p1, hs1, dis, b1r, W2)

    p2 = _agg_kernel(_bf16_table(hs2[:, :DH]), _bf16_table(hs2[:, DH:]),
                     src3, dst3, w3, zrow)

    out = pl.pallas_call(
        _head_body,
        grid=(N // BM,),
        in_specs=[pl.BlockSpec((NC, 2, BM, DH), lambda i: (0, 0, i, 0)),
                  pl.BlockSpec((BM, H), lambda i: (i, 0)),
                  pl.BlockSpec((BM, 1), lambda i: (i, 0)),
                  pl.BlockSpec((1, H), lambda i: (0, 0)),
                  pl.BlockSpec((BM, 1), lambda i: (i, 0)),
                  pl.BlockSpec((2 * H, 1024), lambda i: (0, 0)),
                  pl.BlockSpec((1, 1024), lambda i: (0, 0)),
                  pl.BlockSpec((1024, 1), lambda i: (0, 0)),
                  pl.BlockSpec((1, 1), lambda i: (0, 0))],
        out_specs=pl.BlockSpec((G, 1), lambda i: (0, 0)),
        out_shape=jax.ShapeDtypeStruct((G, 1), f32),
        scratch_shapes=[pltpu.VMEM((G, H), f32),
                        pltpu.VMEM((G, H), f32),
                        pltpu.VMEM((G, H), f32)],
    )(p2, hs2, dis, b2r, batch2, fc1_w, fc1_br, fc2_w, fc2_br)
    return out


# speedup vs baseline: 1.1937x; 1.1191x over previous
import dataclasses
import functools

import jax
import jax.numpy as jnp
from jax import lax
from jax.experimental import pallas as pl
from jax.experimental.pallas import tpu as pltpu
from jax.experimental.pallas import tpu_sc as plsc

N = 10000
E = 320000
D = 128
H = 128
G = 16

NC = 2
NS = 16
NW = NC * NS
ET = E // NW
K = 80
CH = ET // K
NP = 10240
RPT = NP // NS
BM = 1000

_mesh = plsc.VectorSubcoreMesh(
    core_axis_name="c", subcore_axis_name="s", num_cores=NC, num_subcores=NS)

_sc_params = pltpu.CompilerParams(use_tc_tiling_on_sc=False)
if "needs_layout_passes" in pltpu.CompilerParams.__dataclass_fields__:
    _sc_params = dataclasses.replace(_sc_params, needs_layout_passes=False)


def _splat16(v):
    return jnp.full((16,), v, jnp.int32)


@functools.partial(
    pl.kernel,
    out_type=jax.ShapeDtypeStruct((NC, NP, 16), jnp.float32),
    mesh=_mesh,
    scratch_types=[
        pltpu.VMEM((CH, K), jnp.int32),
        pltpu.VMEM((CH, K), jnp.float32),
        pltpu.VMEM((K, 16), jnp.float32),
        pltpu.VMEM_SHARED((NP, 16), jnp.float32),
    ],
    compiler_params=_sc_params,
)
def _deg_kernel(dst_hbm, w_hbm, zer_hbm, out_hbm, dstv, wv, wrow, acc):
    cid = lax.axis_index("c")
    sid = lax.axis_index("s")
    wid = cid * NS + sid
    pltpu.sync_copy(zer_hbm, acc.at[pl.ds(sid * RPT, RPT)])
    pltpu.sync_copy(dst_hbm.at[wid], dstv)
    pltpu.sync_copy(w_hbm.at[wid], wv)
    plsc.subcore_barrier()

    @pl.loop(0, CH)
    def _chunk(j):
        @pl.loop(0, K)
        def _row(i):
            wb = plsc.load_gather(wv, [_splat16(j), _splat16(i)])
            wrow[i, pl.ds(0, 16)] = wb
        pltpu.sync_copy(wrow, acc.at[dstv.at[j]], add=True)

    plsc.subcore_barrier()
    pltpu.sync_copy(acc.at[pl.ds(sid * RPT, RPT)],
                    out_hbm.at[cid, pl.ds(sid * RPT, RPT)])


DH = D // 2

@functools.partial(
    pl.kernel,
    out_type=jax.ShapeDtypeStruct((NC, 2, NP, DH), jnp.float32),
    mesh=_mesh,
    scratch_types=[
        pltpu.VMEM((CH, K), jnp.int32),
        pltpu.VMEM((CH, K), jnp.int32),
        pltpu.VMEM((CH, K), jnp.float32),
        pltpu.VMEM((K, DH), jnp.float32),
        pltpu.VMEM_SHARED((NP, DH), jnp.float32),
    ],
    compiler_params=_sc_params,
)
def _agg_kernel(hs_lo_hbm, hs_hi_hbm, src_hbm, dst_hbm, w_hbm, zer_hbm,
                out_hbm, srcv, dstv, wv, rows, acc):
    cid = lax.axis_index("c")
    sid = lax.axis_index("s")
    wid = cid * NS + sid
    pltpu.sync_copy(src_hbm.at[wid], srcv)
    pltpu.sync_copy(dst_hbm.at[wid], dstv)
    pltpu.sync_copy(w_hbm.at[wid], wv)

    for phase, hs_hbm in enumerate([hs_lo_hbm, hs_hi_hbm]):
        pltpu.sync_copy(zer_hbm, acc.at[pl.ds(sid * RPT, RPT)])
        plsc.subcore_barrier()

        @pl.loop(0, CH)
        def _chunk(j):
            pltpu.sync_copy(hs_hbm.at[srcv.at[j]], rows)
            @pl.loop(0, K)
            def _row(i):
                wb = plsc.load_gather(wv, [_splat16(j), _splat16(i)])
                for s in range(DH // 16):
                    sl = (i, pl.ds(s * 16, 16))
                    rows[sl] = rows[sl] * wb
            pltpu.sync_copy(rows, acc.at[dstv.at[j]], add=True)

        plsc.subcore_barrier()
        pltpu.sync_copy(acc.at[pl.ds(sid * RPT, RPT)],
                        out_hbm.at[cid, phase, pl.ds(sid * RPT, RPT)])


def _mm_body(x_ref, w_ref, o_ref):
    o_ref[...] = jnp.dot(x_ref[...], w_ref[...],
                         preferred_element_type=jnp.float32)


def _dis_scale_body(degp_ref, h_ref, dis_ref, hs_ref):
    dp = degp_ref[...]
    deg = dp[0, :, 0:1] + dp[1, :, 0:1] + 1.0
    dis = lax.rsqrt(deg)
    dis_ref[...] = dis
    hs_ref[...] = h_ref[...] * dis


def _layer2_body(p_ref, hs1_ref, dis_ref, b1_ref, w2_ref, hs2_ref):
    p = p_ref[...]
    ps = p[0] + p[1]
    pcat = jnp.concatenate([ps[0], ps[1]], axis=1)
    dis = dis_ref[...]
    x2 = jnp.maximum((pcat + hs1_ref[...]) * dis + b1_ref[...], 0.0)
    hs2_ref[...] = jnp.dot(x2, w2_ref[...],
                           preferred_element_type=jnp.float32) * dis


def _head_body(p_ref, hs2_ref, dis_ref, b2_ref, batch_ref,
               fc1w_ref, fc1b_ref, fc2w_ref, fc2b_ref, o_ref,
               sum_ref, max_ref, cnt_ref):
    i = pl.program_id(0)

    @pl.when(i == 0)
    def _init():
        sum_ref[...] = jnp.zeros_like(sum_ref)
        max_ref[...] = jnp.full_like(max_ref, -jnp.inf)
        cnt_ref[...] = jnp.zeros_like(cnt_ref)

    p = p_ref[...]
    ps = p[0] + p[1]
    pcat = jnp.concatenate([ps[0], ps[1]], axis=1)
    dis = dis_ref[...]
    x3 = jnp.maximum((pcat + hs2_ref[...]) * dis + b2_ref[...], 0.0)
    bt = batch_ref[...]
    for g in range(G):
        m = bt == g
        cnt_ref[g:g + 1, :] += jnp.sum(m.astype(jnp.float32))
        sum_ref[g:g + 1, :] += jnp.sum(jnp.where(m, x3, 0.0), axis=0,
                                       keepdims=True)
        max_ref[g:g + 1, :] = jnp.maximum(
            max_ref[g:g + 1, :],
            jnp.max(jnp.where(m, x3, -jnp.inf), axis=0, keepdims=True))

    @pl.when(i == N // BM - 1)
    def _fin():
        mean = sum_ref[...] / jnp.maximum(cnt_ref[...], 1.0)
        gf = jnp.concatenate([mean, max_ref[...]], axis=1)
        r = jnp.maximum(jnp.dot(gf, fc1w_ref[...],
                                preferred_element_type=jnp.float32)
                        + fc1b_ref[...], 0.0)
        o_ref[...] = jnp.dot(r, fc2w_ref[...],
                             preferred_element_type=jnp.float32) + fc2b_ref[...]


def kernel(x, edge_index, edge_attr, batch, W1, b1, W2, b2,
           fc1_w, fc1_b, fc2_w, fc2_b):
    f32 = jnp.float32
    src3 = edge_index[0].astype(jnp.int32).reshape(NW, CH, K)
    dst3 = edge_index[1].astype(jnp.int32).reshape(NW, CH, K)
    w3 = edge_attr.astype(f32).reshape(NW, CH, K)
    batch2 = batch.astype(jnp.int32).reshape(N, 1)
    zrow = jnp.zeros((RPT, DH), f32)
    zdeg = jnp.zeros((RPT, 16), f32)

    b1r = b1.reshape(1, H)
    b2r = b2.reshape(1, H)
    fc1_br = fc1_b.reshape(1, 1024)
    fc2_br = fc2_b.reshape(1, 1)

    degp = _deg_kernel(dst3, w3, zdeg)

    h1 = pl.pallas_call(
        _mm_body,
        grid=(N // BM,),
        in_specs=[pl.BlockSpec((BM, D), lambda i: (i, 0)),
                  pl.BlockSpec((D, H), lambda i: (0, 0))],
        out_specs=pl.BlockSpec((BM, H), lambda i: (i, 0)),
        out_shape=jax.ShapeDtypeStruct((N, H), f32),
    )(x, W1)

    dis, hs1 = pl.pallas_call(
        _dis_scale_body,
        grid=(N // BM,),
        in_specs=[pl.BlockSpec((NC, BM, 16), lambda i: (0, i, 0)),
                  pl.BlockSpec((BM, H), lambda i: (i, 0))],
        out_specs=[pl.BlockSpec((BM, 1), lambda i: (i, 0)),
                   pl.BlockSpec((BM, H), lambda i: (i, 0))],
        out_shape=[jax.ShapeDtypeStruct((N, 1), f32),
                   jax.ShapeDtypeStruct((N, H), f32)],
    )(degp, h1)

    p1 = _agg_kernel(hs1[:, :DH], hs1[:, DH:], src3, dst3, w3, zrow)

    hs2 = pl.pallas_call(
        _layer2_body,
        grid=(N // BM,),
        in_specs=[pl.BlockSpec((NC, 2, BM, DH), lambda i: (0, 0, i, 0)),
                  pl.BlockSpec((BM, H), lambda i: (i, 0)),
                  pl.BlockSpec((BM, 1), lambda i: (i, 0)),
                  pl.BlockSpec((1, H), lambda i: (0, 0)),
                  pl.BlockSpec((H, H), lambda i: (0, 0))],
        out_specs=pl.BlockSpec((BM, H), lambda i: (i, 0)),
        out_shape=jax.ShapeDtypeStruct((N, H), f32),
    )(p1, hs1, dis, b1r, W2)

    p2 = _agg_kernel(hs2[:, :DH], hs2[:, DH:], src3, dst3, w3, zrow)

    out = pl.pallas_call(
        _head_body,
        grid=(N // BM,),
        in_specs=[pl.BlockSpec((NC, 2, BM, DH), lambda i: (0, 0, i, 0)),
                  pl.BlockSpec((BM, H), lambda i: (i, 0)),
                  pl.BlockSpec((BM, 1), lambda i: (i, 0)),
                  pl.BlockSpec((1, H), lambda i: (0, 0)),
                  pl.BlockSpec((BM, 1), lambda i: (i, 0)),
                  pl.BlockSpec((2 * H, 1024), lambda i: (0, 0)),
                  pl.BlockSpec((1, 1024), lambda i: (0, 0)),
                  pl.BlockSpec((1024, 1), lambda i: (0, 0)),
                  pl.BlockSpec((1, 1), lambda i: (0, 0))],
        out_specs=pl.BlockSpec((G, 1), lambda i: (0, 0)),
        out_shape=jax.ShapeDtypeStruct((G, 1), f32),
        scratch_shapes=[pltpu.VMEM((G, H), f32),
                        pltpu.VMEM((G, H), f32),
                        pltpu.VMEM((G, H), f32)],
    )(p2, hs2, dis, b2r, batch2, fc1_w, fc1_br, fc2_w, fc2_br)
    return out
